# Initial kernel scaffold; baseline (speedup 1.0000x reference)
import jax
import jax.numpy as jnp
from jax.experimental import pallas as pl


def kernel(x, edge_index, edge_attr, conv1_w, conv1_b, conv2_w, conv2_b, conv3_w, conv3_b, fc1_w, fc1_b, fc2_w, fc2_b):
    raise NotImplementedError("write your pallas kernel here")


# speedup vs baseline: 1.9191x; 1.9191x over previous
import functools

import jax
import jax.numpy as jnp
from jax import lax
from jax.experimental import pallas as pl
from jax.experimental.pallas import tpu as pltpu
from jax.experimental.pallas import tpu_sc as plsc

N = 10000
N_PAD = 10240
E = 320000
E_PAD = 327680
EB = E_PAD // 128
NB_T = E_PAD // (16 * 128)
NB_W = E_PAD // (32 * 128)
NPT = N_PAD // 16
CH = 16
H4 = 4 * N_PAD

_mesh = lambda: plsc.VectorSubcoreMesh(core_axis_name="c", subcore_axis_name="s")


def _zero2d(ref, rows, width):
    z = jnp.zeros((16,), jnp.float32)

    def body(r, _):
        for j in range(width // 16):
            ref[r, pl.ds(j * 16, 16)] = z
        return 0

    lax.fori_loop(0, rows, body, 0)


def _build_dis_tc():
    R = 1024

    def body(d, out):
        dd = d[...]
        out[...] = jnp.where(dd > 0.0, lax.rsqrt(dd), 0.0)

    return pl.pallas_call(
        body,
        grid=(N_PAD // R,),
        in_specs=[pl.BlockSpec((R, 128), lambda i: (i, 0))],
        out_specs=pl.BlockSpec((R, 128), lambda i: (i, 0)),
        out_shape=jax.ShapeDtypeStruct((N_PAD, 128), jnp.float32),
    )


def _build_normrow_kernel():
    @functools.partial(
        pl.kernel,
        out_type=jax.ShapeDtypeStruct((E_PAD, 16), jnp.float32),
        mesh=_mesh(),
        scratch_types=[
            pltpu.VMEM((CH, 128), jnp.int32),
            pltpu.VMEM((CH, 128), jnp.int32),
            pltpu.VMEM((128, 128), jnp.float32),
            pltpu.VMEM((128, 128), jnp.float32),
            pltpu.VMEM((128, 16), jnp.float32),
            pltpu.VMEM((128, 16), jnp.float32),
            pltpu.SemaphoreType.DMA,
            pltpu.SemaphoreType.DMA,
        ],
    )
    def normrow_kernel(row_h, col_h, ew_h, dis_h, norm_h,
                       rowc, colc, drb, dcb, ewb, nb, sem1, sem2):
        cid = lax.axis_index("c")
        sid = lax.axis_index("s")
        wbase = (cid * 16 + sid) * NB_W

        def chunk_body(c, _):
            cb = wbase + c * CH
            pltpu.sync_copy(row_h.at[pl.ds(cb, CH)], rowc)
            pltpu.sync_copy(col_h.at[pl.ds(cb, CH)], colc)

            def body(bb, _):
                cp1 = pltpu.async_copy(dis_h.at[rowc.at[bb]], drb, sem1)
                cp2 = pltpu.async_copy(dis_h.at[colc.at[bb]], dcb, sem2)
                pltpu.sync_copy(ew_h.at[pl.ds((cb + bb) * 128, 128)], ewb)
                cp1.wait()
                cp2.wait()
                for r in range(128):
                    nb[r, :] = (drb[r, pl.ds(0, 16)] * ewb[r, :]
                                * dcb[r, pl.ds(0, 16)])
                pltpu.sync_copy(nb, norm_h.at[pl.ds((cb + bb) * 128, 128)])
                return 0

            lax.fori_loop(0, CH, body, 0)
            return 0

        lax.fori_loop(0, NB_W // CH, chunk_body, 0)

    return normrow_kernel


def _build_prop_kernel():
    @functools.partial(
        pl.kernel,
        out_type=jax.ShapeDtypeStruct((2 * N_PAD, 128), jnp.float32),
        mesh=_mesh(),
        scratch_types=[
            pltpu.VMEM((CH, 128), jnp.int32),
            pltpu.VMEM((CH, 128), jnp.int32),
            pltpu.VMEM((8, 128), jnp.int32),
            pltpu.VMEM((128, 16), jnp.float32),
            pltpu.VMEM((128, 128), jnp.float32),
            pltpu.VMEM((1, 128), jnp.int32),
            pltpu.VMEM_SHARED((N_PAD, 128), jnp.float32),
            pltpu.SemaphoreType.DMA,
        ],
    )
    def prop(h_h, row_h, col_h, norm_h, base_h, out_h,
             rowc, colc, basev, nbuf, buf, adj, acc, sem):
        cid = lax.axis_index("c")
        sid = lax.axis_index("s")

        pltpu.sync_copy(base_h.at[pl.ds(cid * 8, 8)], basev)

        _zero2d(buf, 128, 128)
        for k in range(NPT // 128):
            pltpu.sync_copy(buf, acc.at[pl.ds(sid * NPT + k * 128, 128)])
        plsc.subcore_barrier()

        bv = basev[0, pl.ds(0, 16)]

        def chunk_body(c, _):
            cb = sid * NB_T + c * CH
            pltpu.sync_copy(row_h.at[pl.ds(cb, CH)], rowc)
            pltpu.sync_copy(col_h.at[pl.ds(cb, CH)], colc)

            def edge_body(bb, _):
                for j in range(8):
                    adj[0, pl.ds(j * 16, 16)] = (
                        rowc[bb, pl.ds(j * 16, 16)] + bv)
                cp = pltpu.async_copy(h_h.at[adj.at[0]], buf, sem)
                pltpu.sync_copy(norm_h.at[pl.ds((cb + bb) * 128, 128)], nbuf)
                cp.wait()
                for r in range(128):
                    nr = nbuf[r, :]
                    for j in range(8):
                        buf[r, pl.ds(j * 16, 16)] = (
                            buf[r, pl.ds(j * 16, 16)] * nr)
                pltpu.sync_copy(buf, acc.at[colc.at[bb]], add=True)
                return 0

            lax.fori_loop(0, CH, edge_body, 0)
            return 0

        lax.fori_loop(0, NB_T // CH, chunk_body, 0)
        plsc.subcore_barrier()

        for k in range(NPT // 128):
            pltpu.sync_copy(acc.at[pl.ds(sid * NPT + k * 128, 128)], buf)
            pltpu.sync_copy(
                buf,
                out_h.at[pl.ds(cid * N_PAD + sid * NPT + k * 128, 128)])

    return prop


_dis_tc = _build_dis_tc()
_normrow_kernel = _build_normrow_kernel()
_prop = _build_prop_kernel()


def _elu(a):
    return jnp.where(a > 0.0, a, jnp.exp(jnp.minimum(a, 0.0)) - 1.0)


def _build_layer(C_in, F_out, C_out, flat_out):
    R = 1024
    G = N_PAD // R
    F_in = C_in * 128

    def body(h0, h1, h2, h3, w, b, out):
        hs = (h0, h1, h2, h3)
        acc = jnp.zeros((R, F_out), jnp.float32)
        for k in range(4):
            for c in range(C_in):
                acc += jnp.dot(hs[k][c], w[k, c * 128:(c + 1) * 128, :],
                               preferred_element_type=jnp.float32)
        y = _elu(acc + b[0])
        if flat_out:
            out[...] = y
        else:
            for c in range(C_out):
                out[c] = y[:, c * 128:(c + 1) * 128]

    h_spec = pl.BlockSpec((C_in, R, 128), lambda i: (0, i, 0))
    if flat_out:
        out_shape = jax.ShapeDtypeStruct((N_PAD, F_out), jnp.float32)
        out_spec = pl.BlockSpec((R, F_out), lambda i: (i, 0))
    else:
        out_shape = jax.ShapeDtypeStruct((C_out, N_PAD, 128), jnp.float32)
        out_spec = pl.BlockSpec((C_out, R, 128), lambda i: (0, i, 0))

    return pl.pallas_call(
        body,
        grid=(G,),
        in_specs=[
            h_spec, h_spec, h_spec, h_spec,
            pl.BlockSpec((4, F_in, F_out), lambda i: (0, 0, 0)),
            pl.BlockSpec((1, F_out), lambda i: (0, 0)),
        ],
        out_specs=out_spec,
        out_shape=out_shape,
    )


_layer1 = _build_layer(1, 256, 2, False)
_layer2 = _build_layer(2, 512, 4, False)
_layer3 = _build_layer(4, 1024, 0, True)


def _build_head():
    R = 1024
    G = N_PAD // R

    def body(h, w1, b1, w2, b2, out):
        a = _elu(jnp.dot(h[...], w1[...],
                         preferred_element_type=jnp.float32) + b1[0])
        z = jnp.dot(a, w2[...], preferred_element_type=jnp.float32) + b2[0]
        z = z - jnp.max(z, axis=1, keepdims=True)
        out[...] = z - jnp.log(jnp.sum(jnp.exp(z), axis=1, keepdims=True))

    return pl.pallas_call(
        body,
        grid=(G,),
        in_specs=[
            pl.BlockSpec((R, 1024), lambda i: (i, 0)),
            pl.BlockSpec((1024, 1024), lambda i: (0, 0)),
            pl.BlockSpec((1, 1024), lambda i: (0, 0)),
            pl.BlockSpec((1024, 40), lambda i: (0, 0)),
            pl.BlockSpec((1, 40), lambda i: (0, 0)),
        ],
        out_specs=pl.BlockSpec((R, 40), lambda i: (i, 0)),
        out_shape=jax.ShapeDtypeStruct((N_PAD, 40), jnp.float32),
    )


_head = _build_head()


def _bases(b0, b1):
    top = jnp.full((8, 128), b0, jnp.int32)
    bot = jnp.full((8, 128), b1, jnp.int32)
    return jnp.concatenate([top, bot])


def _pad4(h):
    return jnp.pad(h, ((0, H4 - h.shape[0]), (0, 0)))


def kernel(x, edge_index, edge_attr, conv1_w, conv1_b, conv2_w, conv2_b,
           conv3_w, conv3_b, fc1_w, fc1_b, fc2_w, fc2_b):
    row = edge_index[0]
    col = edge_index[1]
    ew = edge_attr[:, 0]
    extra = E_PAD - E
    rowp = jnp.concatenate([row, jnp.zeros((extra,), jnp.int32)])
    colp = jnp.concatenate([col, jnp.full((extra,), N_PAD - 1, jnp.int32)])
    ewp = jnp.concatenate([ew, jnp.zeros((extra,), jnp.float32)])
    row2d = rowp.reshape(EB, 128)
    col2d = colp.reshape(EB, 128)
    ew_rows = jnp.broadcast_to(ewp[:, None], (E_PAD, 16))

    b00 = _bases(0, 0)
    b01 = _bases(0, N_PAD)
    b23 = _bases(2 * N_PAD, 3 * N_PAD)

    ones = jnp.ones((H4, 128), jnp.float32)
    deg = _prop(ones, row2d, col2d, ew_rows, b00)[:N_PAD]
    dis = _dis_tc(deg)
    norm_rows = _normrow_kernel(row2d, col2d, ew_rows, dis)

    xpad = jnp.pad(x, ((0, N_PAD - N), (0, 0)))

    h1 = _prop(_pad4(xpad), row2d, col2d, norm_rows, b00)[:N_PAD]
    h2 = _prop(_pad4(h1), row2d, col2d, norm_rows, b00)[:N_PAD]
    h3 = _prop(_pad4(h2), row2d, col2d, norm_rows, b00)[:N_PAD]
    y1 = _layer1(xpad[None], h1[None], h2[None], h3[None], conv1_w,
                 conv1_b.reshape(1, 256))

    g0 = y1.reshape(2 * N_PAD, 128)
    g1 = _prop(_pad4(g0), row2d, col2d, norm_rows, b01)
    g2 = _prop(_pad4(g1), row2d, col2d, norm_rows, b01)
    g3 = _prop(_pad4(g2), row2d, col2d, norm_rows, b01)
    y2 = _layer2(y1, g1.reshape(2, N_PAD, 128), g2.reshape(2, N_PAD, 128),
                 g3.reshape(2, N_PAD, 128), conv2_w,
                 conv2_b.reshape(1, 512))

    f0 = y2.reshape(4 * N_PAD, 128)

    def hop4(h):
        a = _prop(h, row2d, col2d, norm_rows, b01)
        b = _prop(h, row2d, col2d, norm_rows, b23)
        return jnp.concatenate([a, b])

    f1 = hop4(f0)
    f2 = hop4(f1)
    f3 = hop4(f2)
    y3 = _layer3(y2, f1.reshape(4, N_PAD, 128), f2.reshape(4, N_PAD, 128),
                 f3.reshape(4, N_PAD, 128), conv3_w,
                 conv3_b.reshape(1, 1024))

    out = _head(y3, fc1_w, fc1_b.reshape(1, 1024), fc2_w,
                fc2_b.reshape(1, 40))
    return out[:N]


# speedup vs baseline: 2.3375x; 1.2180x over previous
import functools

import jax
import jax.numpy as jnp
from jax import lax
from jax.experimental import pallas as pl
from jax.experimental.pallas import tpu as pltpu
from jax.experimental.pallas import tpu_sc as plsc

N = 10000
N_PAD = 10240
E = 320000
E_PAD = 327680
EB = E_PAD // 128
NB_T = E_PAD // (16 * 128)
NB_W = E_PAD // (32 * 128)
NPT = N_PAD // 16
CH = 16
H4 = 4 * N_PAD

_mesh = lambda: plsc.VectorSubcoreMesh(core_axis_name="c", subcore_axis_name="s")


def _zero2d(ref, rows, width):
    z = jnp.zeros((16,), jnp.float32)

    def body(r, _):
        for j in range(width // 16):
            ref[r, pl.ds(j * 16, 16)] = z
        return 0

    lax.fori_loop(0, rows, body, 0)


def _build_dis_tc():
    R = 1024

    def body(d, out):
        dd = d[...]
        out[...] = jnp.where(dd > 0.0, lax.rsqrt(dd), 0.0)

    return pl.pallas_call(
        body,
        grid=(N_PAD // R,),
        in_specs=[pl.BlockSpec((R, 128), lambda i: (i, 0))],
        out_specs=pl.BlockSpec((R, 128), lambda i: (i, 0)),
        out_shape=jax.ShapeDtypeStruct((N_PAD, 128), jnp.float32),
    )


def _build_normrow_kernel():
    @functools.partial(
        pl.kernel,
        out_type=jax.ShapeDtypeStruct((E_PAD, 16), jnp.float32),
        mesh=_mesh(),
        scratch_types=[
            pltpu.VMEM((CH, 128), jnp.int32),
            pltpu.VMEM((CH, 128), jnp.int32),
            pltpu.VMEM((128, 128), jnp.float32),
            pltpu.VMEM((128, 128), jnp.float32),
            pltpu.VMEM((128, 16), jnp.float32),
            pltpu.VMEM((128, 16), jnp.float32),
            pltpu.SemaphoreType.DMA,
            pltpu.SemaphoreType.DMA,
        ],
    )
    def normrow_kernel(row_h, col_h, ew_h, dis_h, norm_h,
                       rowc, colc, drb, dcb, ewb, nb, sem1, sem2):
        cid = lax.axis_index("c")
        sid = lax.axis_index("s")
        wbase = (cid * 16 + sid) * NB_W

        def chunk_body(c, _):
            cb = wbase + c * CH
            pltpu.sync_copy(row_h.at[pl.ds(cb, CH)], rowc)
            pltpu.sync_copy(col_h.at[pl.ds(cb, CH)], colc)

            def body(bb, _):
                cp1 = pltpu.async_copy(dis_h.at[rowc.at[bb]], drb, sem1)
                cp2 = pltpu.async_copy(dis_h.at[colc.at[bb]], dcb, sem2)
                pltpu.sync_copy(ew_h.at[pl.ds((cb + bb) * 128, 128)], ewb)
                cp1.wait()
                cp2.wait()
                for r in range(128):
                    nb[r, :] = (drb[r, pl.ds(0, 16)] * ewb[r, :]
                                * dcb[r, pl.ds(0, 16)])
                pltpu.sync_copy(nb, norm_h.at[pl.ds((cb + bb) * 128, 128)])
                return 0

            lax.fori_loop(0, CH, body, 0)
            return 0

        lax.fori_loop(0, NB_W // CH, chunk_body, 0)

    return normrow_kernel


def _build_prop_kernel():
    @functools.partial(
        pl.kernel,
        out_type=jax.ShapeDtypeStruct((2 * N_PAD, 128), jnp.float32),
        mesh=_mesh(),
        scratch_types=[
            pltpu.VMEM((CH, 128), jnp.int32),
            pltpu.VMEM((CH, 128), jnp.int32),
            pltpu.VMEM((8, 128), jnp.int32),
            pltpu.VMEM((32, 16), jnp.float32),
            pltpu.VMEM((32, 16), jnp.float32),
            pltpu.VMEM((32, 16), jnp.float32),
            pltpu.VMEM((32, 16), jnp.float32),
            pltpu.VMEM((32, 128), jnp.float32),
            pltpu.VMEM((32, 128), jnp.float32),
            pltpu.VMEM((32, 128), jnp.float32),
            pltpu.VMEM((32, 128), jnp.float32),
            pltpu.VMEM((1, 32), jnp.int32),
            pltpu.VMEM((1, 32), jnp.int32),
            pltpu.VMEM((1, 32), jnp.int32),
            pltpu.VMEM((1, 32), jnp.int32),
            pltpu.VMEM((1, 32), jnp.int32),
            pltpu.VMEM((1, 32), jnp.int32),
            pltpu.VMEM((1, 32), jnp.int32),
            pltpu.VMEM((1, 32), jnp.int32),
            pltpu.VMEM_SHARED((N_PAD, 128), jnp.float32),
            pltpu.SemaphoreType.DMA,
            pltpu.SemaphoreType.DMA,
            pltpu.SemaphoreType.DMA,
            pltpu.SemaphoreType.DMA,
            pltpu.SemaphoreType.DMA,
            pltpu.SemaphoreType.DMA,
            pltpu.SemaphoreType.DMA,
            pltpu.SemaphoreType.DMA,
            pltpu.SemaphoreType.DMA,
            pltpu.SemaphoreType.DMA,
            pltpu.SemaphoreType.DMA,
            pltpu.SemaphoreType.DMA,
        ],
    )
    def prop(h_h, row_h, col_h, norm_h, base_h, out_h,
             rowc, colc, basev,
             nb0, nb1, nb2, nb3, fb0, fb1, fb2, fb3,
             ga0, ga1, ga2, ga3, sc0, sc1, sc2, sc3,
             acc,
             g0, g1, g2, g3, n0, n1, n2, n3, s0, s1, s2, s3):
        nbufs = (nb0, nb1, nb2, nb3)
        bufs = (fb0, fb1, fb2, fb3)
        adjs = (ga0, ga1, ga2, ga3)
        cidx = (sc0, sc1, sc2, sc3)
        gsem = (g0, g1, g2, g3)
        nsem = (n0, n1, n2, n3)
        ssem = (s0, s1, s2, s3)

        def wait_g(q):
            pltpu.make_async_copy(
                h_h.at[adjs[q].at[0]], bufs[q], gsem[q]).wait()

        def wait_n(q):
            pltpu.make_async_copy(
                norm_h.at[pl.ds(0, 32)], nbufs[q], nsem[q]).wait()

        def wait_s(q):
            pltpu.make_async_copy(
                h_h.at[adjs[q].at[0]], bufs[q], ssem[q]).wait()

        cid = lax.axis_index("c")
        sid = lax.axis_index("s")

        pltpu.sync_copy(base_h.at[pl.ds(cid * 8, 8)], basev)

        zi = jnp.zeros((16,), jnp.int32)
        for q in range(4):
            _zero2d(bufs[q], 32, 128)
            cidx[q][0, pl.ds(0, 16)] = zi
            cidx[q][0, pl.ds(16, 16)] = zi

        for k in range(NPT // 32):
            pltpu.sync_copy(bufs[0], acc.at[pl.ds(sid * NPT + k * 32, 32)])

        for q in range(4):
            pltpu.async_copy(bufs[q], acc.at[cidx[q].at[0]], ssem[q],
                             add=True)
        plsc.subcore_barrier()

        bv = basev[0, pl.ds(0, 16)]

        def chunk_body(c, _):
            cb = sid * NB_T + c * CH
            pltpu.sync_copy(row_h.at[pl.ds(cb, CH)], rowc)
            pltpu.sync_copy(col_h.at[pl.ds(cb, CH)], colc)

            for m in range(3):
                wait_s(m)
                for j in range(2):
                    adjs[m][0, pl.ds(j * 16, 16)] = (
                        rowc[0, pl.ds(m * 32 + j * 16, 16)] + bv)
                pltpu.async_copy(h_h.at[adjs[m].at[0]], bufs[m], gsem[m])
                pltpu.async_copy(norm_h.at[pl.ds(cb * 128 + m * 32, 32)],
                                 nbufs[m], nsem[m])

            def row_body(r, _):
                for q in range(4):
                    wait_g(q)
                    wait_n(q)
                    for r2 in range(32):
                        nr = nbufs[q][r2, :]
                        for j in range(8):
                            bufs[q][r2, pl.ds(j * 16, 16)] = (
                                bufs[q][r2, pl.ds(j * 16, 16)] * nr)
                    for j in range(2):
                        cidx[q][0, pl.ds(j * 16, 16)] = colc[
                            r, pl.ds(q * 32 + j * 16, 16)]
                    pltpu.async_copy(bufs[q], acc.at[cidx[q].at[0]],
                                     ssem[q], add=True)

                    c2 = (q + 3) % 4
                    if q == 0:
                        wait_s(c2)
                        for j in range(2):
                            adjs[c2][0, pl.ds(j * 16, 16)] = (
                                rowc[r, pl.ds(96 + j * 16, 16)] + bv)
                        pltpu.async_copy(h_h.at[adjs[c2].at[0]], bufs[c2],
                                         gsem[c2])
                        pltpu.async_copy(
                            norm_h.at[pl.ds((cb + r) * 128 + 96, 32)],
                            nbufs[c2], nsem[c2])
                    else:
                        @pl.when(r < CH - 1)
                        def _():
                            wait_s(c2)
                            for j in range(2):
                                adjs[c2][0, pl.ds(j * 16, 16)] = (
                                    rowc[r + 1,
                                         pl.ds((q - 1) * 32 + j * 16, 16)]
                                    + bv)
                            pltpu.async_copy(h_h.at[adjs[c2].at[0]],
                                             bufs[c2], gsem[c2])
                            pltpu.async_copy(
                                norm_h.at[pl.ds(
                                    (cb + r + 1) * 128 + (q - 1) * 32, 32)],
                                nbufs[c2], nsem[c2])
                return 0

            lax.fori_loop(0, CH, row_body, 0)
            return 0

        lax.fori_loop(0, NB_T // CH, chunk_body, 0)

        for q in range(4):
            wait_s(q)
        plsc.subcore_barrier()

        for k in range(NPT // 32):
            pltpu.sync_copy(acc.at[pl.ds(sid * NPT + k * 32, 32)], bufs[0])
            pltpu.sync_copy(
                bufs[0],
                out_h.at[pl.ds(cid * N_PAD + sid * NPT + k * 32, 32)])

    return prop


_dis_tc = _build_dis_tc()
_normrow_kernel = _build_normrow_kernel()
_prop = _build_prop_kernel()


def _elu(a):
    return jnp.where(a > 0.0, a, jnp.exp(jnp.minimum(a, 0.0)) - 1.0)


def _build_layer(C_in, F_out, C_out, flat_out):
    R = 1024
    G = N_PAD // R
    F_in = C_in * 128

    def body(h0, h1, h2, h3, w, b, out):
        hs = (h0, h1, h2, h3)
        acc = jnp.zeros((R, F_out), jnp.float32)
        for k in range(4):
            for c in range(C_in):
                acc += jnp.dot(hs[k][c], w[k, c * 128:(c + 1) * 128, :],
                               preferred_element_type=jnp.float32)
        y = _elu(acc + b[0])
        if flat_out:
            out[...] = y
        else:
            for c in range(C_out):
                out[c] = y[:, c * 128:(c + 1) * 128]

    h_spec = pl.BlockSpec((C_in, R, 128), lambda i: (0, i, 0))
    if flat_out:
        out_shape = jax.ShapeDtypeStruct((N_PAD, F_out), jnp.float32)
        out_spec = pl.BlockSpec((R, F_out), lambda i: (i, 0))
    else:
        out_shape = jax.ShapeDtypeStruct((C_out, N_PAD, 128), jnp.float32)
        out_spec = pl.BlockSpec((C_out, R, 128), lambda i: (0, i, 0))

    return pl.pallas_call(
        body,
        grid=(G,),
        in_specs=[
            h_spec, h_spec, h_spec, h_spec,
            pl.BlockSpec((4, F_in, F_out), lambda i: (0, 0, 0)),
            pl.BlockSpec((1, F_out), lambda i: (0, 0)),
        ],
        out_specs=out_spec,
        out_shape=out_shape,
    )


_layer1 = _build_layer(1, 256, 2, False)
_layer2 = _build_layer(2, 512, 4, False)
_layer3 = _build_layer(4, 1024, 0, True)


def _build_head():
    R = 1024
    G = N_PAD // R

    def body(h, w1, b1, w2, b2, out):
        a = _elu(jnp.dot(h[...], w1[...],
                         preferred_element_type=jnp.float32) + b1[0])
        z = jnp.dot(a, w2[...], preferred_element_type=jnp.float32) + b2[0]
        z = z - jnp.max(z, axis=1, keepdims=True)
        out[...] = z - jnp.log(jnp.sum(jnp.exp(z), axis=1, keepdims=True))

    return pl.pallas_call(
        body,
        grid=(G,),
        in_specs=[
            pl.BlockSpec((R, 1024), lambda i: (i, 0)),
            pl.BlockSpec((1024, 1024), lambda i: (0, 0)),
            pl.BlockSpec((1, 1024), lambda i: (0, 0)),
            pl.BlockSpec((1024, 40), lambda i: (0, 0)),
            pl.BlockSpec((1, 40), lambda i: (0, 0)),
        ],
        out_specs=pl.BlockSpec((R, 40), lambda i: (i, 0)),
        out_shape=jax.ShapeDtypeStruct((N_PAD, 40), jnp.float32),
    )


_head = _build_head()


def _bases(b0, b1):
    top = jnp.full((8, 128), b0, jnp.int32)
    bot = jnp.full((8, 128), b1, jnp.int32)
    return jnp.concatenate([top, bot])


def _pad4(h):
    return jnp.pad(h, ((0, H4 - h.shape[0]), (0, 0)))


def kernel(x, edge_index, edge_attr, conv1_w, conv1_b, conv2_w, conv2_b,
           conv3_w, conv3_b, fc1_w, fc1_b, fc2_w, fc2_b):
    row = edge_index[0]
    col = edge_index[1]
    ew = edge_attr[:, 0]
    extra = E_PAD - E
    rowp = jnp.concatenate([row, jnp.zeros((extra,), jnp.int32)])
    colp = jnp.concatenate([col, jnp.full((extra,), N_PAD - 1, jnp.int32)])
    ewp = jnp.concatenate([ew, jnp.zeros((extra,), jnp.float32)])
    row2d = rowp.reshape(EB, 128)
    col2d = colp.reshape(EB, 128)
    ew_rows = jnp.broadcast_to(ewp[:, None], (E_PAD, 16))

    b00 = _bases(0, 0)
    b01 = _bases(0, N_PAD)
    b23 = _bases(2 * N_PAD, 3 * N_PAD)

    ones = jnp.ones((H4, 128), jnp.float32)
    deg = _prop(ones, row2d, col2d, ew_rows, b00)[:N_PAD]
    dis = _dis_tc(deg)
    norm_rows = _normrow_kernel(row2d, col2d, ew_rows, dis)

    xpad = jnp.pad(x, ((0, N_PAD - N), (0, 0)))

    h1 = _prop(_pad4(xpad), row2d, col2d, norm_rows, b00)[:N_PAD]
    h2 = _prop(_pad4(h1), row2d, col2d, norm_rows, b00)[:N_PAD]
    h3 = _prop(_pad4(h2), row2d, col2d, norm_rows, b00)[:N_PAD]
    y1 = _layer1(xpad[None], h1[None], h2[None], h3[None], conv1_w,
                 conv1_b.reshape(1, 256))

    g0 = y1.reshape(2 * N_PAD, 128)
    g1 = _prop(_pad4(g0), row2d, col2d, norm_rows, b01)
    g2 = _prop(_pad4(g1), row2d, col2d, norm_rows, b01)
    g3 = _prop(_pad4(g2), row2d, col2d, norm_rows, b01)
    y2 = _layer2(y1, g1.reshape(2, N_PAD, 128), g2.reshape(2, N_PAD, 128),
                 g3.reshape(2, N_PAD, 128), conv2_w,
                 conv2_b.reshape(1, 512))

    f0 = y2.reshape(4 * N_PAD, 128)

    def hop4(h):
        a = _prop(h, row2d, col2d, norm_rows, b01)
        b = _prop(h, row2d, col2d, norm_rows, b23)
        return jnp.concatenate([a, b])

    f1 = hop4(f0)
    f2 = hop4(f1)
    f3 = hop4(f2)
    y3 = _layer3(y2, f1.reshape(4, N_PAD, 128), f2.reshape(4, N_PAD, 128),
                 f3.reshape(4, N_PAD, 128), conv3_w,
                 conv3_b.reshape(1, 1024))

    out = _head(y3, fc1_w, fc1_b.reshape(1, 1024), fc2_w,
                fc2_b.reshape(1, 40))
    return out[:N]


# speedup vs baseline: 2.5832x; 1.1051x over previous
import functools

import jax
import jax.numpy as jnp
from jax import lax
from jax.experimental import pallas as pl
from jax.experimental.pallas import tpu as pltpu
from jax.experimental.pallas import tpu_sc as plsc

N = 10000
N_PAD = 10240
E = 320000
E_PAD = 327680
EB = E_PAD // 128
NB_T = E_PAD // (16 * 128)
NB_W = E_PAD // (32 * 128)
NPT = N_PAD // 16
CH = 16
H4 = 4 * N_PAD

_mesh = lambda: plsc.VectorSubcoreMesh(core_axis_name="c", subcore_axis_name="s")


def _zero2d(ref, rows, width):
    z = jnp.zeros((16,), jnp.float32)

    def body(r, _):
        for j in range(width // 16):
            ref[r, pl.ds(j * 16, 16)] = z
        return 0

    lax.fori_loop(0, rows, body, 0)


def _build_dis_tc():
    R = 1024

    def body(d, o1, o2):
        dd = d[...]
        r = jnp.where(dd > 0.0, lax.rsqrt(dd), 0.0)
        o1[...] = r
        o2[...] = r * r

    return pl.pallas_call(
        body,
        grid=(N_PAD // R,),
        in_specs=[pl.BlockSpec((R, 128), lambda i: (i, 0))],
        out_specs=[pl.BlockSpec((R, 128), lambda i: (i, 0)),
                   pl.BlockSpec((R, 128), lambda i: (i, 0))],
        out_shape=[jax.ShapeDtypeStruct((N_PAD, 128), jnp.float32),
                   jax.ShapeDtypeStruct((N_PAD, 128), jnp.float32)],
    )


def _build_scale(C):
    R = 1024

    def body(h, t, out):
        out[...] = h[...] * t[...]

    return pl.pallas_call(
        body,
        grid=(C, N_PAD // R),
        in_specs=[pl.BlockSpec((1, R, 128), lambda c, i: (c, i, 0)),
                  pl.BlockSpec((R, 128), lambda c, i: (i, 0))],
        out_specs=pl.BlockSpec((1, R, 128), lambda c, i: (c, i, 0)),
        out_shape=jax.ShapeDtypeStruct((C, N_PAD, 128), jnp.float32),
    )


def _build_deg_kernel():
    @functools.partial(
        pl.kernel,
        out_type=jax.ShapeDtypeStruct((N_PAD, 128), jnp.float32),
        mesh=_mesh(),
        scratch_types=[
            pltpu.VMEM((CH, 128), jnp.int32),
            pltpu.VMEM((128, 16), jnp.float32),
            pltpu.VMEM((128, 128), jnp.float32),
            pltpu.VMEM_SHARED((N_PAD, 128), jnp.float32),
        ],
    )
    def deg_kernel(col_h, ew_h, deg_h, colc, ewb, web, acc):
        cid = lax.axis_index("c")
        sid = lax.axis_index("s")

        _zero2d(web, 128, 128)
        for k in range(NPT // 128):
            pltpu.sync_copy(web, acc.at[pl.ds(sid * NPT + k * 128, 128)])
        plsc.subcore_barrier()

        def chunk_body(c, _):
            cb = sid * NB_T + c * CH
            pltpu.sync_copy(col_h.at[pl.ds(cb, CH)], colc)

            def body(bb, _):
                pltpu.sync_copy(ew_h.at[pl.ds((cb + bb) * 128, 128)], ewb)
                for r in range(128):
                    er = ewb[r, :]
                    for j in range(8):
                        web[r, pl.ds(j * 16, 16)] = er
                pltpu.sync_copy(web, acc.at[colc.at[bb]], add=True)
                return 0

            lax.fori_loop(0, CH, body, 0)
            return 0

        lax.fori_loop(0, NB_T // CH, chunk_body, 0)
        plsc.subcore_barrier()

        off = cid * (N_PAD // 2) + sid * (NPT // 2)
        for k in range(NPT // 2 // 64):
            pltpu.sync_copy(acc.at[pl.ds(off + k * 64, 64)],
                            web.at[pl.ds(0, 64)])
            pltpu.sync_copy(web.at[pl.ds(0, 64)],
                            deg_h.at[pl.ds(off + k * 64, 64)])

    return deg_kernel


def _build_prop_kernel():
    @functools.partial(
        pl.kernel,
        out_type=jax.ShapeDtypeStruct((2 * N_PAD, 128), jnp.float32),
        mesh=_mesh(),
        scratch_types=[
            pltpu.VMEM((CH, 128), jnp.int32),
            pltpu.VMEM((CH, 128), jnp.int32),
            pltpu.VMEM((8, 128), jnp.int32),
            pltpu.VMEM((32, 16), jnp.float32),
            pltpu.VMEM((32, 16), jnp.float32),
            pltpu.VMEM((32, 16), jnp.float32),
            pltpu.VMEM((32, 16), jnp.float32),
            pltpu.VMEM((32, 128), jnp.float32),
            pltpu.VMEM((32, 128), jnp.float32),
            pltpu.VMEM((32, 128), jnp.float32),
            pltpu.VMEM((32, 128), jnp.float32),
            pltpu.VMEM((1, 32), jnp.int32),
            pltpu.VMEM((1, 32), jnp.int32),
            pltpu.VMEM((1, 32), jnp.int32),
            pltpu.VMEM((1, 32), jnp.int32),
            pltpu.VMEM((1, 32), jnp.int32),
            pltpu.VMEM((1, 32), jnp.int32),
            pltpu.VMEM((1, 32), jnp.int32),
            pltpu.VMEM((1, 32), jnp.int32),
            pltpu.VMEM_SHARED((N_PAD, 128), jnp.float32),
            pltpu.SemaphoreType.DMA,
            pltpu.SemaphoreType.DMA,
            pltpu.SemaphoreType.DMA,
            pltpu.SemaphoreType.DMA,
            pltpu.SemaphoreType.DMA,
            pltpu.SemaphoreType.DMA,
            pltpu.SemaphoreType.DMA,
            pltpu.SemaphoreType.DMA,
            pltpu.SemaphoreType.DMA,
            pltpu.SemaphoreType.DMA,
            pltpu.SemaphoreType.DMA,
            pltpu.SemaphoreType.DMA,
        ],
    )
    def prop(h_h, row_h, col_h, norm_h, base_h, out_h,
             rowc, colc, basev,
             nb0, nb1, nb2, nb3, fb0, fb1, fb2, fb3,
             ga0, ga1, ga2, ga3, sc0, sc1, sc2, sc3,
             acc,
             g0, g1, g2, g3, n0, n1, n2, n3, s0, s1, s2, s3):
        nbufs = (nb0, nb1, nb2, nb3)
        bufs = (fb0, fb1, fb2, fb3)
        adjs = (ga0, ga1, ga2, ga3)
        cidx = (sc0, sc1, sc2, sc3)
        gsem = (g0, g1, g2, g3)
        nsem = (n0, n1, n2, n3)
        ssem = (s0, s1, s2, s3)

        def wait_g(q):
            pltpu.make_async_copy(
                h_h.at[adjs[q].at[0]], bufs[q], gsem[q]).wait()

        def wait_n(q):
            pltpu.make_async_copy(
                norm_h.at[pl.ds(0, 32)], nbufs[q], nsem[q]).wait()

        def wait_s(q):
            pltpu.make_async_copy(
                h_h.at[adjs[q].at[0]], bufs[q], ssem[q]).wait()

        cid = lax.axis_index("c")
        sid = lax.axis_index("s")

        pltpu.sync_copy(base_h.at[pl.ds(cid * 8, 8)], basev)

        zi = jnp.zeros((16,), jnp.int32)
        for q in range(4):
            _zero2d(bufs[q], 32, 128)
            cidx[q][0, pl.ds(0, 16)] = zi
            cidx[q][0, pl.ds(16, 16)] = zi

        for k in range(NPT // 32):
            pltpu.sync_copy(bufs[0], acc.at[pl.ds(sid * NPT + k * 32, 32)])

        for q in range(4):
            pltpu.async_copy(bufs[q], acc.at[cidx[q].at[0]], ssem[q],
                             add=True)
        plsc.subcore_barrier()

        bv = basev[0, pl.ds(0, 16)]

        def chunk_body(c, _):
            cb = sid * NB_T + c * CH
            pltpu.sync_copy(row_h.at[pl.ds(cb, CH)], rowc)
            pltpu.sync_copy(col_h.at[pl.ds(cb, CH)], colc)

            for m in range(3):
                wait_s(m)
                for j in range(2):
                    adjs[m][0, pl.ds(j * 16, 16)] = (
                        rowc[0, pl.ds(m * 32 + j * 16, 16)] + bv)
                pltpu.async_copy(h_h.at[adjs[m].at[0]], bufs[m], gsem[m])
                pltpu.async_copy(norm_h.at[pl.ds(cb * 128 + m * 32, 32)],
                                 nbufs[m], nsem[m])

            def row_body(r, _):
                for q in range(4):
                    wait_g(q)
                    wait_n(q)
                    for r2 in range(32):
                        nr = nbufs[q][r2, :]
                        for j in range(8):
                            bufs[q][r2, pl.ds(j * 16, 16)] = (
                                bufs[q][r2, pl.ds(j * 16, 16)] * nr)
                    for j in range(2):
                        cidx[q][0, pl.ds(j * 16, 16)] = colc[
                            r, pl.ds(q * 32 + j * 16, 16)]
                    pltpu.async_copy(bufs[q], acc.at[cidx[q].at[0]],
                                     ssem[q], add=True)

                    c2 = (q + 3) % 4
                    if q == 0:
                        wait_s(c2)
                        for j in range(2):
                            adjs[c2][0, pl.ds(j * 16, 16)] = (
                                rowc[r, pl.ds(96 + j * 16, 16)] + bv)
                        pltpu.async_copy(h_h.at[adjs[c2].at[0]], bufs[c2],
                                         gsem[c2])
                        pltpu.async_copy(
                            norm_h.at[pl.ds((cb + r) * 128 + 96, 32)],
                            nbufs[c2], nsem[c2])
                    else:
                        @pl.when(r < CH - 1)
                        def _():
                            wait_s(c2)
                            for j in range(2):
                                adjs[c2][0, pl.ds(j * 16, 16)] = (
                                    rowc[r + 1,
                                         pl.ds((q - 1) * 32 + j * 16, 16)]
                                    + bv)
                            pltpu.async_copy(h_h.at[adjs[c2].at[0]],
                                             bufs[c2], gsem[c2])
                            pltpu.async_copy(
                                norm_h.at[pl.ds(
                                    (cb + r + 1) * 128 + (q - 1) * 32, 32)],
                                nbufs[c2], nsem[c2])
                return 0

            lax.fori_loop(0, CH, row_body, 0)
            return 0

        lax.fori_loop(0, NB_T // CH, chunk_body, 0)

        for q in range(4):
            wait_s(q)
        plsc.subcore_barrier()

        for k in range(NPT // 32):
            pltpu.sync_copy(acc.at[pl.ds(sid * NPT + k * 32, 32)], bufs[0])
            pltpu.sync_copy(
                bufs[0],
                out_h.at[pl.ds(cid * N_PAD + sid * NPT + k * 32, 32)])

    return prop


_deg_kernel = _build_deg_kernel()
_dis_tc = _build_dis_tc()
_scale1 = _build_scale(1)
_scale2 = _build_scale(2)
_scale4 = _build_scale(4)
_prop = _build_prop_kernel()


def _elu(a):
    return jnp.where(a > 0.0, a, jnp.exp(jnp.minimum(a, 0.0)) - 1.0)


def _build_layer(C_in, F_out, C_out, flat_out):
    R = 1024
    G = N_PAD // R
    F_in = C_in * 128

    def body(h0, h1, h2, h3, dis, w, b, out):
        hs = (h0, h1, h2, h3)
        d = dis[...]
        acc = jnp.zeros((R, F_out), jnp.float32)
        for k in range(4):
            for c in range(C_in):
                blk = hs[k][c]
                if k > 0:
                    blk = blk * d
                acc += jnp.dot(blk, w[k, c * 128:(c + 1) * 128, :],
                               preferred_element_type=jnp.float32)
        y = _elu(acc + b[0])
        if flat_out:
            out[...] = y
        else:
            for c in range(C_out):
                out[c] = y[:, c * 128:(c + 1) * 128]

    h_spec = pl.BlockSpec((C_in, R, 128), lambda i: (0, i, 0))
    if flat_out:
        out_shape = jax.ShapeDtypeStruct((N_PAD, F_out), jnp.float32)
        out_spec = pl.BlockSpec((R, F_out), lambda i: (i, 0))
    else:
        out_shape = jax.ShapeDtypeStruct((C_out, N_PAD, 128), jnp.float32)
        out_spec = pl.BlockSpec((C_out, R, 128), lambda i: (0, i, 0))

    return pl.pallas_call(
        body,
        grid=(G,),
        in_specs=[
            h_spec, h_spec, h_spec, h_spec,
            pl.BlockSpec((R, 128), lambda i: (i, 0)),
            pl.BlockSpec((4, F_in, F_out), lambda i: (0, 0, 0)),
            pl.BlockSpec((1, F_out), lambda i: (0, 0)),
        ],
        out_specs=out_spec,
        out_shape=out_shape,
    )


_layer1 = _build_layer(1, 256, 2, False)
_layer2 = _build_layer(2, 512, 4, False)
_layer3 = _build_layer(4, 1024, 0, True)


def _build_head():
    R = 1024
    G = N_PAD // R

    def body(h, w1, b1, w2, b2, out):
        a = _elu(jnp.dot(h[...], w1[...],
                         preferred_element_type=jnp.float32) + b1[0])
        z = jnp.dot(a, w2[...], preferred_element_type=jnp.float32) + b2[0]
        z = z - jnp.max(z, axis=1, keepdims=True)
        out[...] = z - jnp.log(jnp.sum(jnp.exp(z), axis=1, keepdims=True))

    return pl.pallas_call(
        body,
        grid=(G,),
        in_specs=[
            pl.BlockSpec((R, 1024), lambda i: (i, 0)),
            pl.BlockSpec((1024, 1024), lambda i: (0, 0)),
            pl.BlockSpec((1, 1024), lambda i: (0, 0)),
            pl.BlockSpec((1024, 40), lambda i: (0, 0)),
            pl.BlockSpec((1, 40), lambda i: (0, 0)),
        ],
        out_specs=pl.BlockSpec((R, 40), lambda i: (i, 0)),
        out_shape=jax.ShapeDtypeStruct((N_PAD, 40), jnp.float32),
    )


_head = _build_head()


def _bases(b0, b1):
    top = jnp.full((8, 128), b0, jnp.int32)
    bot = jnp.full((8, 128), b1, jnp.int32)
    return jnp.concatenate([top, bot])


def _pad4(h):
    return jnp.pad(h, ((0, H4 - h.shape[0]), (0, 0)))


def kernel(x, edge_index, edge_attr, conv1_w, conv1_b, conv2_w, conv2_b,
           conv3_w, conv3_b, fc1_w, fc1_b, fc2_w, fc2_b):
    row = edge_index[0]
    col = edge_index[1]
    ew = edge_attr[:, 0]
    extra = E_PAD - E
    rowp = jnp.concatenate([row, jnp.zeros((extra,), jnp.int32)])
    colp = jnp.concatenate([col, jnp.full((extra,), N_PAD - 1, jnp.int32)])
    ewp = jnp.concatenate([ew, jnp.zeros((extra,), jnp.float32)])
    row2d = rowp.reshape(EB, 128)
    col2d = colp.reshape(EB, 128)
    ew_rows = jnp.broadcast_to(ewp[:, None], (E_PAD, 16))

    b00 = _bases(0, 0)
    b01 = _bases(0, N_PAD)
    b23 = _bases(2 * N_PAD, 3 * N_PAD)

    deg = _deg_kernel(col2d, ew_rows)
    dis, dis2 = _dis_tc(deg)

    xpad = jnp.pad(x, ((0, N_PAD - N), (0, 0)))

    u0 = _scale1(xpad[None], dis).reshape(N_PAD, 128)
    s1 = _prop(_pad4(u0), row2d, col2d, ew_rows, b00)[:N_PAD]
    u1 = _scale1(s1[None], dis2).reshape(N_PAD, 128)
    s2 = _prop(_pad4(u1), row2d, col2d, ew_rows, b00)[:N_PAD]
    u2 = _scale1(s2[None], dis2).reshape(N_PAD, 128)
    s3 = _prop(_pad4(u2), row2d, col2d, ew_rows, b00)[:N_PAD]
    y1 = _layer1(xpad[None], s1[None], s2[None], s3[None], dis, conv1_w,
                 conv1_b.reshape(1, 256))

    v0 = _scale2(y1, dis).reshape(2 * N_PAD, 128)
    g1 = _prop(_pad4(v0), row2d, col2d, ew_rows, b01)
    v1 = _scale2(g1.reshape(2, N_PAD, 128), dis2).reshape(2 * N_PAD, 128)
    g2 = _prop(_pad4(v1), row2d, col2d, ew_rows, b01)
    v2 = _scale2(g2.reshape(2, N_PAD, 128), dis2).reshape(2 * N_PAD, 128)
    g3 = _prop(_pad4(v2), row2d, col2d, ew_rows, b01)
    y2 = _layer2(y1, g1.reshape(2, N_PAD, 128), g2.reshape(2, N_PAD, 128),
                 g3.reshape(2, N_PAD, 128), dis, conv2_w,
                 conv2_b.reshape(1, 512))

    def hop4(h):
        a = _prop(h, row2d, col2d, ew_rows, b01)
        b = _prop(h, row2d, col2d, ew_rows, b23)
        return jnp.concatenate([a, b])

    w0 = _scale4(y2, dis).reshape(H4, 128)
    f1 = hop4(w0)
    w1 = _scale4(f1.reshape(4, N_PAD, 128), dis2).reshape(H4, 128)
    f2 = hop4(w1)
    w2 = _scale4(f2.reshape(4, N_PAD, 128), dis2).reshape(H4, 128)
    f3 = hop4(w2)
    y3 = _layer3(y2, f1.reshape(4, N_PAD, 128), f2.reshape(4, N_PAD, 128),
                 f3.reshape(4, N_PAD, 128), dis, conv3_w,
                 conv3_b.reshape(1, 1024))

    out = _head(y3, fc1_w, fc1_b.reshape(1, 1024), fc2_w,
                fc2_b.reshape(1, 40))
    return out[:N]
